# baseline (speedup 1.0000x reference)
import jax
import jax.numpy as jnp
from jax.experimental import pallas as pl

_N = 10000
_D = 128
_BLOCK = 400


def _fused_mlp_kernel(x_ref, w_emb_ref, w_out_ref, o_ref):
    x = x_ref[...]
    h = jax.lax.dot_general(
        x, w_emb_ref[...], (((1,), (1,)), ((), ())),
        preferred_element_type=jnp.float32,
    )
    h = jnp.maximum(h, 0.0)
    o_ref[...] = jax.lax.dot_general(
        h, w_out_ref[...], (((1,), (1,)), ((), ())),
        preferred_element_type=jnp.float32,
    )


def kernel(features, edge_index, W_emb, W_out, betas):
    del edge_index, betas
    return pl.pallas_call(
        _fused_mlp_kernel,
        grid=(_N // _BLOCK,),
        in_specs=[
            pl.BlockSpec((_BLOCK, _D), lambda i: (i, 0)),
            pl.BlockSpec((_D, _D), lambda i: (0, 0)),
            pl.BlockSpec((_D, _D), lambda i: (0, 0)),
        ],
        out_specs=pl.BlockSpec((_BLOCK, _D), lambda i: (i, 0)),
        out_shape=jax.ShapeDtypeStruct((_N, _D), jnp.float32),
    )(features, W_emb, W_out)


# speedup vs baseline: 2.3862x; 2.3862x over previous
import jax
import jax.numpy as jnp
from jax.experimental import pallas as pl

_N = 10000
_D = 128
_BLOCK = 2000


def _fused_mlp_kernel(x_ref, w_emb_ref, w_out_ref, o_ref):
    x = x_ref[...]
    h = jax.lax.dot_general(
        x, w_emb_ref[...], (((1,), (1,)), ((), ())),
        preferred_element_type=jnp.float32,
    )
    h = jnp.maximum(h, 0.0)
    o_ref[...] = jax.lax.dot_general(
        h, w_out_ref[...], (((1,), (1,)), ((), ())),
        preferred_element_type=jnp.float32,
    )


def kernel(features, edge_index, W_emb, W_out, betas):
    del edge_index, betas
    return pl.pallas_call(
        _fused_mlp_kernel,
        grid=(_N // _BLOCK,),
        in_specs=[
            pl.BlockSpec((_BLOCK, _D), lambda i: (i, 0)),
            pl.BlockSpec((_D, _D), lambda i: (0, 0)),
            pl.BlockSpec((_D, _D), lambda i: (0, 0)),
        ],
        out_specs=pl.BlockSpec((_BLOCK, _D), lambda i: (i, 0)),
        out_shape=jax.ShapeDtypeStruct((_N, _D), jnp.float32),
    )(features, W_emb, W_out)


# speedup vs baseline: 2.7771x; 1.1638x over previous
import jax
import jax.numpy as jnp
from jax.experimental import pallas as pl

_N = 10000
_D = 128
_BLOCK = 5000


def _fused_mlp_kernel(x_ref, w_emb_ref, w_out_ref, o_ref):
    x = x_ref[...]
    h = jax.lax.dot_general(
        x, w_emb_ref[...], (((1,), (1,)), ((), ())),
        preferred_element_type=jnp.float32,
    )
    h = jnp.maximum(h, 0.0)
    o_ref[...] = jax.lax.dot_general(
        h, w_out_ref[...], (((1,), (1,)), ((), ())),
        preferred_element_type=jnp.float32,
    )


def kernel(features, edge_index, W_emb, W_out, betas):
    del edge_index, betas
    return pl.pallas_call(
        _fused_mlp_kernel,
        grid=(_N // _BLOCK,),
        in_specs=[
            pl.BlockSpec((_BLOCK, _D), lambda i: (i, 0)),
            pl.BlockSpec((_D, _D), lambda i: (0, 0)),
            pl.BlockSpec((_D, _D), lambda i: (0, 0)),
        ],
        out_specs=pl.BlockSpec((_BLOCK, _D), lambda i: (i, 0)),
        out_shape=jax.ShapeDtypeStruct((_N, _D), jnp.float32),
    )(features, W_emb, W_out)


# speedup vs baseline: 2.7849x; 1.0028x over previous
import jax
import jax.numpy as jnp
from jax.experimental import pallas as pl

_N = 10000
_D = 128
_BLOCK = 5000


def _fused_mlp_kernel(x_ref, w_emb_ref, w_out_ref, o_ref):
    x = x_ref[...].astype(jnp.bfloat16)
    h = jax.lax.dot_general(
        x, w_emb_ref[...].astype(jnp.bfloat16), (((1,), (1,)), ((), ())),
        preferred_element_type=jnp.float32,
    )
    h = jnp.maximum(h, 0.0).astype(jnp.bfloat16)
    o_ref[...] = jax.lax.dot_general(
        h, w_out_ref[...].astype(jnp.bfloat16), (((1,), (1,)), ((), ())),
        preferred_element_type=jnp.float32,
    )


def kernel(features, edge_index, W_emb, W_out, betas):
    del edge_index, betas
    return pl.pallas_call(
        _fused_mlp_kernel,
        grid=(_N // _BLOCK,),
        in_specs=[
            pl.BlockSpec((_BLOCK, _D), lambda i: (i, 0)),
            pl.BlockSpec((_D, _D), lambda i: (0, 0)),
            pl.BlockSpec((_D, _D), lambda i: (0, 0)),
        ],
        out_specs=pl.BlockSpec((_BLOCK, _D), lambda i: (i, 0)),
        out_shape=jax.ShapeDtypeStruct((_N, _D), jnp.float32),
    )(features, W_emb, W_out)
